# baseline (speedup 1.0000x reference)
import jax
import jax.numpy as jnp
from jax.experimental import pallas as pl
from jax.experimental.pallas import tpu as pltpu

E = 2048
B = 64
BJ = 256
BB = 8
NL = 128


def _bits_kernel(x_ref, out_ref):
    v = 2 * x_ref[:, :].T + 1
    out_ref[:, :, : E - NL] = jnp.zeros((BJ, BB, E - NL), jnp.float32)
    shifts = jnp.minimum(
        (NL - 1) - jax.lax.broadcasted_iota(jnp.int32, (BJ, BB, NL), 2), 31
    )
    bits = (v[:, :, None] >> shifts) & 1
    out_ref[:, :, E - NL :] = bits.astype(jnp.float32)


def kernel(x):
    y = pl.pallas_call(
        _bits_kernel,
        grid=(E // BJ, B // BB),
        in_specs=[pl.BlockSpec((BB, BJ), lambda j, i: (i, j))],
        out_specs=pl.BlockSpec((BJ, BB, E), lambda j, i: (j, i, 0)),
        out_shape=jax.ShapeDtypeStruct((E, B, E), jnp.float32),
        compiler_params=pltpu.CompilerParams(
            dimension_semantics=("parallel", "parallel")
        ),
    )(x)
    return jnp.transpose(y, (0, 2, 1))


# speedup vs baseline: 1.0206x; 1.0206x over previous
import jax
import jax.numpy as jnp
from jax.experimental import pallas as pl
from jax.experimental.pallas import tpu as pltpu

E = 2048
B = 64
BJ = 128
BB = 8
NL = 128


def _bits_kernel(x_ref, out_ref):
    v = 2 * x_ref[:, :].T + 1
    out_ref[:, :, : E - NL] = jnp.zeros((BJ, BB, E - NL), jnp.float32)
    shifts = jnp.minimum(
        (NL - 1) - jax.lax.broadcasted_iota(jnp.int32, (BJ, BB, NL), 2), 31
    )
    bits = (v[:, :, None] >> shifts) & 1
    out_ref[:, :, E - NL :] = bits.astype(jnp.float32)


def kernel(x):
    y = pl.pallas_call(
        _bits_kernel,
        grid=(E // BJ, B // BB),
        in_specs=[pl.BlockSpec((BB, BJ), lambda j, i: (i, j))],
        out_specs=pl.BlockSpec((BJ, BB, E), lambda j, i: (j, i, 0)),
        out_shape=jax.ShapeDtypeStruct((E, B, E), jnp.float32),
        compiler_params=pltpu.CompilerParams(
            dimension_semantics=("parallel", "parallel")
        ),
    )(x)
    return jnp.transpose(y, (0, 2, 1))
